# baseline (speedup 1.0000x reference)
import functools

import jax
import jax.numpy as jnp
from jax import lax
from jax.experimental import pallas as pl
from jax.experimental.pallas import tpu as pltpu
from jax.experimental.pallas import tpu_sc as plsc

B = 16384
F = 26
M = 16

_NROWS = 1000000
_NC = 2
_NS = 16
_NW = _NC * _NS
_SPW = B // _NW
_C = 128
_NCHUNK = _SPW // _C
_IPC = _C * F
_IPW = _SPW * F
_GL = 128

_NFULL = _NROWS // 128
_TAILW = _NROWS - _NFULL * 128
_MW = 768
_NMACRO = _NFULL * 128 // _MW
_NI = 42
_NSMALL = F + _NCHUNK * F
_SPI = 4


def _relayout_body(w2t_hbm, tail_hbm, idxt_hbm, w1_hbm,
                   out_hbm, lin_hbm, idx_out_hbm,
                   tt0, tt1, ob0, ob1, idx_vm, w1f, linbuf,
                   si0, si1, so0, so1, sw, sx):
    cid = lax.axis_index("c")
    sid = lax.axis_index("s")
    wid = sid * _NC + cid
    lanes = lax.iota(jnp.int32, 16)
    tts, obs, sis, sos = (tt0, tt1), (ob0, ob1), (si0, si1), (so0, so1)

    pltpu.sync_copy(idxt_hbm.at[:, pl.ds(wid * _SPW, _SPW)], idx_vm)

    def in_cp(m, k):
        return pltpu.make_async_copy(
            w2t_hbm.at[:, pl.ds(m * _MW, _MW)], tts[k], sis[k])

    def out_cp(m, k):
        return pltpu.make_async_copy(
            obs[k], out_hbm.at[pl.ds(m * _MW * M, _MW * M)], sos[k])

    def small_cp(gi):
        def idx_cp():
            f = gi
            pltpu.async_copy(
                idx_vm.at[f],
                idx_out_hbm.at[pl.ds((wid * F + f) * _SPW, _SPW)], sx)

        def w1_cp():
            f = (gi - F) // _NCHUNK
            c = (gi - F) % _NCHUNK
            sl = pl.ds(c * _C, _C)
            pltpu.async_copy(w1_hbm.at[idx_vm.at[f, sl]],
                             w1f.at[f, sl], sw)

        lax.cond(gi < F, idx_cp, w1_cp)

    @pl.when(wid < _NMACRO)
    def _():
        in_cp(wid, 0).start()

    def j_body(j, carry):
        for k in (0, 1):
            i = 2 * j + k
            m = wid + i * _NW
            mn = wid + (i + 1) * _NW

            for u in range(_SPI):
                gi = i * _SPI + u

                @pl.when(gi < _NSMALL)
                def _():
                    small_cp(gi)

            @pl.when(mn < _NMACRO)
            def _():
                in_cp(mn, (k + 1) % 2).start()

            @pl.when(m < _NMACRO)
            def _():
                in_cp(m, k).wait()

                @pl.when(j >= 1)
                def _():
                    out_cp(m, k).wait()

                tt, ob = tts[k], obs[k]
                biota = lanes * M

                @plsc.parallel_loop(0, _MW // 16, unroll=8)
                def _(cj):
                    c16 = cj * 16
                    for mrow in range(M):
                        v = tt[mrow, pl.ds(c16, 16)]
                        plsc.store_scatter(
                            ob, [biota + (c16 * M + mrow)], v)

                out_cp(m, k).start()
        return carry

    lax.fori_loop(0, _NI // 2, j_body, 0)
    for k in (0, 1):
        out_cp(0, k).wait()

    @pl.when(wid == _NFULL % _NW)
    def _():
        pltpu.async_copy(tail_hbm, ob0.at[pl.ds(0, _TAILW * M)], si0).wait()
        pltpu.async_copy(ob0.at[pl.ds(0, _TAILW * M)],
                         out_hbm.at[pl.ds(_NFULL * 128 * M, _TAILW * M)],
                         so0).wait()

    for f in range(F):
        pltpu.make_async_copy(
            idx_vm.at[f],
            idx_out_hbm.at[pl.ds((wid * F + f) * _SPW, _SPW)], sx).wait()
    for f in range(F):
        for c in range(_NCHUNK):
            sl = pl.ds(c * _C, _C)
            pltpu.make_async_copy(w1_hbm.at[idx_vm.at[f, sl]],
                                  w1f.at[f, sl], sw).wait()

    @plsc.parallel_loop(0, _SPW // 16, unroll=2)
    def _(g2):
        acc = w1f[0, pl.ds(g2 * 16, 16)]
        for f in range(1, F):
            acc = acc + w1f[f, pl.ds(g2 * 16, 16)]
        linbuf[pl.ds(g2 * 16, 16)] = acc

    pltpu.sync_copy(linbuf, lin_hbm.at[pl.ds(wid * _SPW, _SPW)])


def _fm_body(idx_hbm, lin_hbm, w2_hbm, out_hbm,
             idx_v, linv, rows0, rows1, out0, out1, tbuf,
             sg0, sg1, so0, so1):
    cid = lax.axis_index("c")
    sid = lax.axis_index("s")
    wid = sid * _NC + cid
    base = wid * _SPW
    lanes = lax.iota(jnp.int32, 16)
    rows, sgs, outs, sos = (rows0, rows1), (sg0, sg1), (out0, out1), (so0, so1)

    pltpu.sync_copy(idx_hbm.at[pl.ds(wid * _IPW, _IPW)], idx_v)
    pltpu.sync_copy(lin_hbm.at[pl.ds(base, _SPW)], linv)

    def g_cp(c, f, k):
        sl = pl.ds(f * _SPW + c * _C, _C)
        return pltpu.make_async_copy(
            w2_hbm.at[idx_v.at[sl]], rows[k].at[pl.ds(f * _C, _C)], sgs[k])

    def stage(c, k):
        for f in range(F):
            g_cp(c, f, k).start()

    def out_cp(c, k):
        return pltpu.make_async_copy(
            outs[k], out_hbm.at[pl.ds(base + c * _C, _C)], sos[k])

    stage(0, 0)
    for c in range(_NCHUNK):
        k = c % 2
        if c + 1 < _NCHUNK:
            stage(c + 1, (k + 1) % 2)
        for f in range(F):
            g_cp(c, f, k).wait()
        if c >= 2:
            out_cp(c - 2, k).wait()
        rv, ov = rows[k], outs[k]

        @plsc.parallel_loop(0, _C, unroll=2)
        def _(s):
            r0 = rv[s]
            acc_s = r0
            acc_q = r0 * r0
            for f in range(1, F):
                r = rv[f * _C + s]
                acc_s = acc_s + r
                acc_q = acc_q + r * r
            tbuf[pl.ds(s * 16, 16)] = acc_s * acc_s - acc_q

        @plsc.parallel_loop(0, _C // 16)
        def _(g):
            pbase = (g * 16 + lanes) * 16
            pair = plsc.load_gather(tbuf, [pbase])
            for mm in range(1, 16):
                pair = pair + plsc.load_gather(tbuf, [pbase + mm])
            lin = linv[pl.ds(c * _C + g * 16, 16)]
            ov[pl.ds(g * 16, 16)] = lin + 0.5 * pair

        out_cp(c, k).start()
    for c in (_NCHUNK - 2, _NCHUNK - 1):
        out_cp(c, c % 2).wait()


@jax.jit
def kernel(indices, w_1st, w_2nd):
    idxt = indices.T.astype(jnp.int32)
    mesh = plsc.VectorSubcoreMesh(core_axis_name="c", subcore_axis_name="s")

    relayout = pl.kernel(
        _relayout_body,
        out_type=(
            jax.ShapeDtypeStruct((_NROWS * M,), jnp.float32),
            jax.ShapeDtypeStruct((B,), jnp.float32),
            jax.ShapeDtypeStruct((B * F,), jnp.int32),
        ),
        mesh=mesh,
        compiler_params=pltpu.CompilerParams(
            needs_layout_passes=False, use_tc_tiling_on_sc=True),
        scratch_types=[
            pltpu.VMEM((16, _MW), jnp.float32),
            pltpu.VMEM((16, _MW), jnp.float32),
            pltpu.VMEM((_MW * M,), jnp.float32),
            pltpu.VMEM((_MW * M,), jnp.float32),
            pltpu.VMEM((F, _SPW), jnp.int32),
            pltpu.VMEM((F, _SPW), jnp.float32),
            pltpu.VMEM((_SPW,), jnp.float32),
            pltpu.SemaphoreType.DMA,
            pltpu.SemaphoreType.DMA,
            pltpu.SemaphoreType.DMA,
            pltpu.SemaphoreType.DMA,
            pltpu.SemaphoreType.DMA,
            pltpu.SemaphoreType.DMA,
        ],
    )
    tail = w_2nd[_NFULL * 128:_NROWS].reshape(-1)
    table_flat, linear, idx_lin = relayout(w_2nd.T, tail, idxt, w_1st)
    w2_rm = table_flat.reshape(_NROWS, M)

    fm = pl.kernel(
        _fm_body,
        out_type=jax.ShapeDtypeStruct((B,), jnp.float32),
        mesh=mesh,
        compiler_params=pltpu.CompilerParams(
            needs_layout_passes=False, use_tc_tiling_on_sc=False),
        scratch_types=[
            pltpu.VMEM((_IPW,), jnp.int32),
            pltpu.VMEM((_SPW,), jnp.float32),
            pltpu.VMEM((_IPC, M), jnp.float32),
            pltpu.VMEM((_IPC, M), jnp.float32),
            pltpu.VMEM((_C,), jnp.float32),
            pltpu.VMEM((_C,), jnp.float32),
            pltpu.VMEM((_C * 16,), jnp.float32),
            pltpu.SemaphoreType.DMA,
            pltpu.SemaphoreType.DMA,
            pltpu.SemaphoreType.DMA,
            pltpu.SemaphoreType.DMA,
        ],
    )
    return fm(idx_lin, linear, w2_rm)


# speedup vs baseline: 1.5367x; 1.5367x over previous
import functools

import jax
import jax.numpy as jnp
from jax import lax
from jax.experimental import pallas as pl
from jax.experimental.pallas import tpu as pltpu
from jax.experimental.pallas import tpu_sc as plsc

B = 16384
F = 26
M = 16

_NROWS = 1000000
_NC = 2
_NS = 16
_NW = _NC * _NS
_SPW = B // _NW
_C = 128
_NCHUNK = _SPW // _C
_IPC = _C * F
_IPW = _SPW * F
_GL = 128

_NFULL = _NROWS // 128
_TAILW = _NROWS - _NFULL * 128
_MW = 768
_NMACRO = _NFULL * 128 // _MW
_NI = 42
_NSMALL = F + _NCHUNK * F
_SPI = 4


def _relayout_body(w2t_hbm, tail_hbm, idxt_hbm, w1_hbm,
                   out_hbm, lin_hbm, idx_out_hbm,
                   tt0, tt1, ob0, ob1, idx_vm, w1f, linbuf,
                   si0, si1, so0, so1, sw, sx):
    cid = lax.axis_index("c")
    sid = lax.axis_index("s")
    wid = sid * _NC + cid
    lanes = lax.iota(jnp.int32, 16)
    tts, obs, sis, sos = (tt0, tt1), (ob0, ob1), (si0, si1), (so0, so1)

    pltpu.sync_copy(idxt_hbm.at[:, pl.ds(wid * _SPW, _SPW)], idx_vm)

    def in_cp(m, k):
        return pltpu.make_async_copy(
            w2t_hbm.at[:, pl.ds(m * _MW, _MW)], tts[k], sis[k])

    def out_cp(m, k):
        return pltpu.make_async_copy(
            obs[k], out_hbm.at[pl.ds(m * _MW * M, _MW * M)], sos[k])

    def small_cp(gi):
        def idx_cp():
            f = gi
            pltpu.async_copy(
                idx_vm.at[f],
                idx_out_hbm.at[pl.ds((wid * F + f) * _SPW, _SPW)], sx)

        def w1_cp():
            f = (gi - F) // _NCHUNK
            c = (gi - F) % _NCHUNK
            sl = pl.ds(c * _C, _C)
            pltpu.async_copy(w1_hbm.at[idx_vm.at[f, sl]],
                             w1f.at[f, sl], sw)

        lax.cond(gi < F, idx_cp, w1_cp)

    @pl.when(wid < _NMACRO)
    def _():
        in_cp(wid, 0).start()

    def j_body(j, carry):
        for k in (0, 1):
            i = 2 * j + k
            m = wid + i * _NW
            mn = wid + (i + 1) * _NW

            for u in range(_SPI):
                gi = i * _SPI + u

                @pl.when(gi < _NSMALL)
                def _():
                    small_cp(gi)

            @pl.when(mn < _NMACRO)
            def _():
                in_cp(mn, (k + 1) % 2).start()

            @pl.when(m < _NMACRO)
            def _():
                in_cp(m, k).wait()

                @pl.when(j >= 1)
                def _():
                    out_cp(m, k).wait()

                tt, ob = tts[k], obs[k]
                biota = lanes * M

                @plsc.parallel_loop(0, _MW // 16, unroll=2)
                def _(cj):
                    c16 = cj * 16
                    for mrow in range(M):
                        v = tt[mrow, pl.ds(c16, 16)]
                        plsc.store_scatter(
                            ob, [biota + (c16 * M + mrow)], v)

                out_cp(m, k).start()
        return carry

    lax.fori_loop(0, _NI // 2, j_body, 0)
    for k in (0, 1):
        out_cp(0, k).wait()

    @pl.when(wid == _NFULL % _NW)
    def _():
        pltpu.async_copy(tail_hbm, ob0.at[pl.ds(0, _TAILW * M)], si0).wait()
        pltpu.async_copy(ob0.at[pl.ds(0, _TAILW * M)],
                         out_hbm.at[pl.ds(_NFULL * 128 * M, _TAILW * M)],
                         so0).wait()

    for f in range(F):
        pltpu.make_async_copy(
            idx_vm.at[f],
            idx_out_hbm.at[pl.ds((wid * F + f) * _SPW, _SPW)], sx).wait()
    for f in range(F):
        for c in range(_NCHUNK):
            sl = pl.ds(c * _C, _C)
            pltpu.make_async_copy(w1_hbm.at[idx_vm.at[f, sl]],
                                  w1f.at[f, sl], sw).wait()

    @plsc.parallel_loop(0, _SPW // 16, unroll=2)
    def _(g2):
        acc = w1f[0, pl.ds(g2 * 16, 16)]
        for f in range(1, F):
            acc = acc + w1f[f, pl.ds(g2 * 16, 16)]
        linbuf[pl.ds(g2 * 16, 16)] = acc

    pltpu.sync_copy(linbuf, lin_hbm.at[pl.ds(wid * _SPW, _SPW)])


def _fm_body(idx_hbm, lin_hbm, w2_hbm, out_hbm,
             idx_v, linv, rows0, rows1, out0, out1, tbuf,
             sg0, sg1, so0, so1):
    cid = lax.axis_index("c")
    sid = lax.axis_index("s")
    wid = sid * _NC + cid
    base = wid * _SPW
    lanes = lax.iota(jnp.int32, 16)
    rows, sgs, outs, sos = (rows0, rows1), (sg0, sg1), (out0, out1), (so0, so1)

    pltpu.sync_copy(idx_hbm.at[pl.ds(wid * _IPW, _IPW)], idx_v)
    pltpu.sync_copy(lin_hbm.at[pl.ds(base, _SPW)], linv)

    def g_cp(c, f, k):
        sl = pl.ds(f * _SPW + c * _C, _C)
        return pltpu.make_async_copy(
            w2_hbm.at[idx_v.at[sl]], rows[k].at[pl.ds(f * _C, _C)], sgs[k])

    def stage(c, k):
        for f in range(F):
            g_cp(c, f, k).start()

    def out_cp(c, k):
        return pltpu.make_async_copy(
            outs[k], out_hbm.at[pl.ds(base + c * _C, _C)], sos[k])

    stage(0, 0)
    for c in range(_NCHUNK):
        k = c % 2
        if c + 1 < _NCHUNK:
            stage(c + 1, (k + 1) % 2)
        for f in range(F):
            g_cp(c, f, k).wait()
        if c >= 2:
            out_cp(c - 2, k).wait()
        rv, ov = rows[k], outs[k]

        @plsc.parallel_loop(0, _C, unroll=2)
        def _(s):
            r0 = rv[s]
            acc_s = r0
            acc_q = r0 * r0
            for f in range(1, F):
                r = rv[f * _C + s]
                acc_s = acc_s + r
                acc_q = acc_q + r * r
            tbuf[pl.ds(s * 16, 16)] = acc_s * acc_s - acc_q

        @plsc.parallel_loop(0, _C // 16)
        def _(g):
            pbase = (g * 16 + lanes) * 16
            pair = plsc.load_gather(tbuf, [pbase])
            for mm in range(1, 16):
                pair = pair + plsc.load_gather(tbuf, [pbase + mm])
            lin = linv[pl.ds(c * _C + g * 16, 16)]
            ov[pl.ds(g * 16, 16)] = lin + 0.5 * pair

        out_cp(c, k).start()
    for c in (_NCHUNK - 2, _NCHUNK - 1):
        out_cp(c, c % 2).wait()


@jax.jit
def kernel(indices, w_1st, w_2nd):
    idxt = indices.T.astype(jnp.int32)
    mesh = plsc.VectorSubcoreMesh(core_axis_name="c", subcore_axis_name="s")

    relayout = pl.kernel(
        _relayout_body,
        out_type=(
            jax.ShapeDtypeStruct((_NROWS * M,), jnp.float32),
            jax.ShapeDtypeStruct((B,), jnp.float32),
            jax.ShapeDtypeStruct((B * F,), jnp.int32),
        ),
        mesh=mesh,
        compiler_params=pltpu.CompilerParams(
            needs_layout_passes=False, use_tc_tiling_on_sc=True),
        scratch_types=[
            pltpu.VMEM((16, _MW), jnp.float32),
            pltpu.VMEM((16, _MW), jnp.float32),
            pltpu.VMEM((_MW * M,), jnp.float32),
            pltpu.VMEM((_MW * M,), jnp.float32),
            pltpu.VMEM((F, _SPW), jnp.int32),
            pltpu.VMEM((F, _SPW), jnp.float32),
            pltpu.VMEM((_SPW,), jnp.float32),
            pltpu.SemaphoreType.DMA,
            pltpu.SemaphoreType.DMA,
            pltpu.SemaphoreType.DMA,
            pltpu.SemaphoreType.DMA,
            pltpu.SemaphoreType.DMA,
            pltpu.SemaphoreType.DMA,
        ],
    )
    tail = w_2nd[_NFULL * 128:_NROWS].reshape(-1)
    table_flat, linear, idx_lin = relayout(w_2nd.T, tail, idxt, w_1st)
    w2_rm = table_flat.reshape(_NROWS, M)

    fm = pl.kernel(
        _fm_body,
        out_type=jax.ShapeDtypeStruct((B,), jnp.float32),
        mesh=mesh,
        compiler_params=pltpu.CompilerParams(
            needs_layout_passes=False, use_tc_tiling_on_sc=False),
        scratch_types=[
            pltpu.VMEM((_IPW,), jnp.int32),
            pltpu.VMEM((_SPW,), jnp.float32),
            pltpu.VMEM((_IPC, M), jnp.float32),
            pltpu.VMEM((_IPC, M), jnp.float32),
            pltpu.VMEM((_C,), jnp.float32),
            pltpu.VMEM((_C,), jnp.float32),
            pltpu.VMEM((_C * 16,), jnp.float32),
            pltpu.SemaphoreType.DMA,
            pltpu.SemaphoreType.DMA,
            pltpu.SemaphoreType.DMA,
            pltpu.SemaphoreType.DMA,
        ],
    )
    return fm(idx_lin, linear, w2_rm)
